# Initial kernel scaffold; baseline (speedup 1.0000x reference)
import jax
import jax.numpy as jnp
from jax.experimental import pallas as pl


def kernel(x, weight):
    raise NotImplementedError("write your pallas kernel here")


# speedup vs baseline: 1.6853x; 1.6853x over previous
import jax
import jax.numpy as jnp
from jax import lax
from jax.experimental import pallas as pl
from jax.experimental.pallas import tpu as pltpu
from jax.experimental.pallas import tpu_sc as plsc

VOCAB = 32000
DIM = 4096
B = 4 * 2048

NUM_CORES = 2
NUM_SUBCORES = 16
NW = NUM_CORES * NUM_SUBCORES
B_PER_W = B // NW
CHUNK = 8
NCHUNK = B_PER_W // CHUNK
NPAIR = NCHUNK // 2


def _emb_body(table_hbm, idx_hbm, out_hbm, idx_v, rows0, rows1,
              gsem0, gsem1, ssem0, ssem1):
    wid = lax.axis_index("s") * NUM_CORES + lax.axis_index("c")
    base = wid * B_PER_W

    pltpu.sync_copy(idx_hbm.at[pl.ds(base, B_PER_W)], idx_v)

    def gather(g, rows, gsem):
        pltpu.async_copy(
            table_hbm.at[idx_v.at[pl.ds(g * CHUNK, CHUNK)]], rows, gsem)

    def put(g, rows, ssem):
        pltpu.async_copy(
            rows, out_hbm.at[pl.ds(base + g * CHUNK, CHUNK)], ssem)

    def wait_gather(rows, gsem):
        pltpu.make_async_copy(
            table_hbm.at[idx_v.at[pl.ds(0, CHUNK)]], rows, gsem).wait()

    def wait_put(rows, ssem):
        pltpu.make_async_copy(
            rows, out_hbm.at[pl.ds(base, CHUNK)], ssem).wait()

    gather(0, rows0, gsem0)
    gather(1, rows1, gsem1)

    def step(h, carry):
        g0 = 2 * h
        g1 = g0 + 1
        wait_gather(rows0, gsem0)
        put(g0, rows0, ssem0)
        wait_gather(rows1, gsem1)
        put(g1, rows1, ssem1)

        @pl.when(h + 1 < NPAIR)
        def _():
            wait_put(rows0, ssem0)
            gather(g0 + 2, rows0, gsem0)
            wait_put(rows1, ssem1)
            gather(g1 + 2, rows1, gsem1)

        return carry

    lax.fori_loop(0, NPAIR, step, 0)

    wait_put(rows0, ssem0)
    wait_put(rows1, ssem1)


@jax.jit
def _embedding_lookup(weight, idx):
    mesh = plsc.VectorSubcoreMesh(
        core_axis_name="c", subcore_axis_name="s",
        num_cores=NUM_CORES, num_subcores=NUM_SUBCORES,
    )
    return pl.kernel(
        _emb_body,
        out_type=jax.ShapeDtypeStruct((B, DIM), jnp.float32),
        mesh=mesh,
        scratch_types=[
            pltpu.VMEM((B_PER_W,), jnp.int32),
            pltpu.VMEM((CHUNK, DIM), jnp.float32),
            pltpu.VMEM((CHUNK, DIM), jnp.float32),
            pltpu.SemaphoreType.DMA,
            pltpu.SemaphoreType.DMA,
            pltpu.SemaphoreType.DMA,
            pltpu.SemaphoreType.DMA,
        ],
    )(weight, idx)


def kernel(x, weight):
    idx = x.reshape(-1).astype(jnp.int32)
    out = _embedding_lookup(weight, idx)
    return out.reshape(x.shape + (DIM,))


# speedup vs baseline: 1.7092x; 1.0142x over previous
import jax
import jax.numpy as jnp
from jax import lax
from jax.experimental import pallas as pl
from jax.experimental.pallas import tpu as pltpu
from jax.experimental.pallas import tpu_sc as plsc

VOCAB = 32000
DIM = 4096
B = 4 * 2048

NUM_CORES = 2
NUM_SUBCORES = 16
NW = NUM_CORES * NUM_SUBCORES
B_PER_W = B // NW
CHUNK = 8
NCHUNK = B_PER_W // CHUNK
NBUF = 3


def _emb_body(table_hbm, idx_hbm, out_hbm, idx_v, rows, gsems, ssems):
    wid = lax.axis_index("s") * NUM_CORES + lax.axis_index("c")
    base = wid * B_PER_W

    pltpu.sync_copy(idx_hbm.at[pl.ds(base, B_PER_W)], idx_v)

    def gather(g, s):
        pltpu.async_copy(
            table_hbm.at[idx_v.at[pl.ds(g * CHUNK, CHUNK)]],
            rows[s], gsems[s])

    def put(g, s):
        pltpu.async_copy(
            rows[s], out_hbm.at[pl.ds(base + g * CHUNK, CHUNK)], ssems[s])

    def wait_gather(s):
        pltpu.make_async_copy(
            table_hbm.at[idx_v.at[pl.ds(0, CHUNK)]], rows[s], gsems[s]).wait()

    def wait_put(s):
        pltpu.make_async_copy(
            rows[s], out_hbm.at[pl.ds(base, CHUNK)], ssems[s]).wait()

    for g in range(NBUF):
        gather(g, g % NBUF)

    for g in range(NCHUNK):
        s = g % NBUF
        wait_gather(s)
        put(g, s)
        gn = g + 2
        if NBUF <= gn < NCHUNK:
            sn = gn % NBUF
            wait_put(sn)
            gather(gn, sn)

    for g in range(NCHUNK - NBUF, NCHUNK):
        wait_put(g % NBUF)


@jax.jit
def _embedding_lookup(weight, idx):
    mesh = plsc.VectorSubcoreMesh(
        core_axis_name="c", subcore_axis_name="s",
        num_cores=NUM_CORES, num_subcores=NUM_SUBCORES,
    )
    return pl.kernel(
        _emb_body,
        out_type=jax.ShapeDtypeStruct((B, DIM), jnp.float32),
        mesh=mesh,
        scratch_types=[
            pltpu.VMEM((B_PER_W,), jnp.int32),
            [pltpu.VMEM((CHUNK, DIM), jnp.float32) for _ in range(NBUF)],
            [pltpu.SemaphoreType.DMA for _ in range(NBUF)],
            [pltpu.SemaphoreType.DMA for _ in range(NBUF)],
        ],
    )(weight, idx)


def kernel(x, weight):
    idx = x.reshape(-1).astype(jnp.int32)
    out = _embedding_lookup(weight, idx)
    return out.reshape(x.shape + (DIM,))
